# baseline (speedup 1.0000x reference)
import functools

import jax
import jax.numpy as jnp
from jax import lax
from jax.experimental import pallas as pl
from jax.experimental.pallas import tpu as pltpu
from jax.experimental.pallas import tpu_sc as plsc

NUM_EMB = 1000000
DIM = 64
B = 16384
N = 819200

NC = 2
NS = 16
NW = NC * NS
BPW = B // NW
C = 640
IB = 128
CB = C // IB
OPAD = 640
PBLK = 16384
NPB = (NUM_EMB + 2 * PBLK - 1) // (2 * PBLK)
NSTAGE = NPB * PBLK


def _sc_sums(sparse_features, offsets_pad, table):
    mesh = plsc.VectorSubcoreMesh(core_axis_name="c", subcore_axis_name="s")

    @functools.partial(
        pl.kernel,
        out_type=jax.ShapeDtypeStruct((B, DIM), jnp.float32),
        mesh=mesh,
        scratch_types=[
            pltpu.VMEM((OPAD,), jnp.int32),
            pltpu.VMEM((2, C), jnp.int32),
            pltpu.VMEM((2, C), jnp.int32),
            pltpu.VMEM((2, C, DIM), jnp.float32),
            pltpu.VMEM_SHARED((B // NC + 8, DIM), jnp.float32),
            pltpu.SemaphoreType.DMA,
            pltpu.SemaphoreType.DMA,
            pltpu.SemaphoreType.DMA,
        ],
        compiler_params=pltpu.CompilerParams(
            needs_layout_passes=False, use_tc_tiling_on_sc=False),
    )
    def k(sf_hbm, off_hbm, table_hbm, out_hbm, loff, idx, seg, rows, acc,
          sem_idx, sem_gat, sem_sca):
        cid = lax.axis_index("c")
        sid = lax.axis_index("s")
        lbase = sid * BPW
        gbase = (cid * NS + sid) * BPW
        trash = B // NC

        pltpu.sync_copy(off_hbm.at[pl.ds(gbase, OPAD)], loff)

        def zrow(r, _):
            for cc in range(DIM // 16):
                rows[0, r, pl.ds(cc * 16, 16)] = jnp.zeros((16,), jnp.float32)
            return 0
        lax.fori_loop(0, IB, zrow, 0)
        for blk in range(BPW // IB):
            pltpu.sync_copy(rows.at[0, pl.ds(0, IB)],
                            acc.at[pl.ds(lbase + blk * IB, IB)])

        iota16 = lax.iota(jnp.int32, 16)
        rtzero = jnp.minimum(iota16, 0)

        t_start = jnp.min(loff[pl.ds(0, 16)])
        t_end = jnp.min(loff[pl.ds(BPW, 16)])
        k0 = t_start // C
        k1 = (t_end + C - 1) // C

        @pl.when(k0 < k1)
        def _prime():
            pltpu.async_copy(sf_hbm.at[pl.ds(k0 * C, C)], idx.at[k0 & 1],
                             sem_idx)

        def chunk(g, _):
            p = g & 1
            s = g * C

            pltpu.make_async_copy(sf_hbm.at[pl.ds(0, C)], idx.at[p],
                                  sem_idx).wait()

            def remap(m, _):
                v = idx[p, pl.ds(m * 16, 16)]
                q = ((v >> 15) << 14) + (v & 16383)
                idx[p, pl.ds(m * 16, 16)] = 2 * q + ((v >> 14) & 1)
                return 0
            lax.fori_loop(0, C // 16, remap, 0)

            @pl.when(g > k0 + 1)
            def _drain_prev():
                pltpu.make_async_copy(rows.at[p], acc.at[seg.at[p]],
                                      sem_sca).wait()

            get = pltpu.async_copy(table_hbm.at[idx.at[p]], rows.at[p],
                                   sem_gat)

            @pl.when(g + 1 < k1)
            def _next_idx():
                pltpu.async_copy(sf_hbm.at[pl.ds((g + 1) * C, C)],
                                 idx.at[1 - p], sem_idx)

            def bsearch(m, _):
                pos = iota16 + (s + m * 16)
                res = rtzero
                step = BPW
                while step >= 1:
                    probe = res + step
                    pidx = jnp.minimum(probe, BPW + 1) - 1
                    v = plsc.load_gather(loff, [pidx])
                    take = (probe <= BPW + 1) & (v <= pos)
                    res = jnp.where(take, probe, res)
                    step //= 2
                lrow = res - 1
                out_of_range = (lrow < 0) | (lrow >= BPW)
                row = jnp.where(out_of_range, trash, lbase + lrow)
                seg[p, pl.ds(m * 16, 16)] = row
                return 0
            lax.fori_loop(0, C // 16, bsearch, 0)

            get.wait()
            pltpu.async_copy(rows.at[p], acc.at[seg.at[p]], sem_sca, add=True)
            return 0

        lax.fori_loop(k0, k1, chunk, 0)

        @pl.when(k1 > k0)
        def _drain_last():
            pltpu.make_async_copy(rows.at[(k1 - 1) & 1],
                                  acc.at[seg.at[(k1 - 1) & 1]],
                                  sem_sca).wait()

        @pl.when(k1 > k0 + 1)
        def _drain_last2():
            pltpu.make_async_copy(rows.at[k1 & 1], acc.at[seg.at[k1 & 1]],
                                  sem_sca).wait()

        for blk in range(BPW // IB):
            pltpu.sync_copy(acc.at[pl.ds(lbase + blk * IB, IB)],
                            out_hbm.at[pl.ds(gbase + blk * IB, IB)])

    return k(sparse_features, offsets_pad, table)


def _tc_pack(table_t):
    def body(lo_ref, hi_ref, o_ref):
        o_ref[:, 0:DIM] = lo_ref[...].T
        o_ref[:, DIM:2 * DIM] = hi_ref[...].T

    return pl.pallas_call(
        body,
        grid=(NPB,),
        in_specs=[
            pl.BlockSpec((DIM, PBLK), lambda i: (0, 2 * i)),
            pl.BlockSpec((DIM, PBLK),
                         lambda i: (0, jnp.minimum(2 * i + 1,
                                                   (NUM_EMB - 1) // PBLK))),
        ],
        out_specs=pl.BlockSpec((PBLK, 2 * DIM), lambda i: (i, 0)),
        out_shape=jax.ShapeDtypeStruct((NSTAGE, 2 * DIM), jnp.float32),
    )(table_t, table_t)


def _tc_transpose(table_t):
    blk = 2048

    def body(t_ref, o_ref):
        o_ref[...] = t_ref[...].T.reshape(blk // 2, 128)

    return pl.pallas_call(
        body,
        grid=((NUM_EMB + blk - 1) // blk,),
        in_specs=[pl.BlockSpec((DIM, blk), lambda i: (0, i))],
        out_specs=pl.BlockSpec((blk // 2, 128), lambda i: (i, 0)),
        out_shape=jax.ShapeDtypeStruct((NUM_EMB // 2, 2 * DIM), jnp.float32),
    )(table_t)


def _tc_finish(sums, offs0, offs1, wt_pad, b_pad):
    blk = 2048

    def body(s_ref, o0_ref, o1_ref, w_ref, b_ref, o_ref):
        cnt = jnp.maximum((o1_ref[...] - o0_ref[...]).astype(jnp.float32), 1.0)
        pooled = s_ref[...] * (1.0 / cnt)[:, None]
        o_ref[...] = (
            jnp.dot(pooled, w_ref[...], preferred_element_type=jnp.float32)
            + b_ref[...]
        )

    return pl.pallas_call(
        body,
        grid=(B // blk,),
        in_specs=[
            pl.BlockSpec((blk, DIM), lambda i: (i, 0)),
            pl.BlockSpec((blk,), lambda i: (i,)),
            pl.BlockSpec((blk,), lambda i: (i,)),
            pl.BlockSpec((DIM, 8), lambda i: (0, 0)),
            pl.BlockSpec((1, 8), lambda i: (0, 0)),
        ],
        out_specs=pl.BlockSpec((blk, 8), lambda i: (i, 0)),
        out_shape=jax.ShapeDtypeStruct((B, 8), jnp.float32),
    )(sums, offs0, offs1, wt_pad, b_pad)


def kernel(sparse_features, offsets, send_shape, table, W, b):
    offsets_pad = jnp.pad(offsets, (0, OPAD - 1), mode="edge")
    staged = _tc_pack(table.T).reshape(2 * NSTAGE, DIM)
    sums = _sc_sums(sparse_features, offsets_pad, staged)
    wt_pad = jnp.zeros((DIM, 8), jnp.float32).at[:, :5].set(W.T)
    b_pad = jnp.zeros((1, 8), jnp.float32).at[0, :5].set(b)
    out8 = _tc_finish(sums, offsets[:B], offsets[1:B + 1], wt_pad, b_pad)
    return out8[:, :5]


# speedup vs baseline: 1.0027x; 1.0027x over previous
import functools

import jax
import jax.numpy as jnp
from jax import lax
from jax.experimental import pallas as pl
from jax.experimental.pallas import tpu as pltpu
from jax.experimental.pallas import tpu_sc as plsc

NUM_EMB = 1000000
DIM = 64
B = 16384
N = 819200

NC = 2
NS = 16
NW = NC * NS
BPW = B // NW
C = 512
IB = 128
CB = C // IB
OPAD = 640
PBLK = 16384
NPB = (NUM_EMB + 2 * PBLK - 1) // (2 * PBLK)
NSTAGE = NPB * PBLK


def _sc_sums(sparse_features, offsets_pad, table):
    mesh = plsc.VectorSubcoreMesh(core_axis_name="c", subcore_axis_name="s")

    @functools.partial(
        pl.kernel,
        out_type=jax.ShapeDtypeStruct((B, DIM), jnp.float32),
        mesh=mesh,
        scratch_types=[
            pltpu.VMEM((OPAD,), jnp.int32),
            pltpu.VMEM((2, C), jnp.int32),
            pltpu.VMEM((2, C), jnp.int32),
            pltpu.VMEM((2, C, DIM), jnp.float32),
            pltpu.VMEM_SHARED((B // NC + 8, DIM), jnp.float32),
            pltpu.SemaphoreType.DMA,
            pltpu.SemaphoreType.DMA,
            pltpu.SemaphoreType.DMA,
        ],
        compiler_params=pltpu.CompilerParams(
            needs_layout_passes=False, use_tc_tiling_on_sc=False),
    )
    def k(sf_hbm, off_hbm, table_hbm, out_hbm, loff, idx, seg, rows, acc,
          sem_idx, sem_gat, sem_sca):
        cid = lax.axis_index("c")
        sid = lax.axis_index("s")
        lbase = sid * BPW
        gbase = (cid * NS + sid) * BPW
        trash = B // NC

        pltpu.sync_copy(off_hbm.at[pl.ds(gbase, OPAD)], loff)

        def zrow(r, _):
            for cc in range(DIM // 16):
                rows[0, r, pl.ds(cc * 16, 16)] = jnp.zeros((16,), jnp.float32)
            return 0
        lax.fori_loop(0, IB, zrow, 0)
        for blk in range(BPW // IB):
            pltpu.sync_copy(rows.at[0, pl.ds(0, IB)],
                            acc.at[pl.ds(lbase + blk * IB, IB)])

        iota16 = lax.iota(jnp.int32, 16)
        rtzero = jnp.minimum(iota16, 0)

        t_start = jnp.min(loff[pl.ds(0, 16)])
        t_end = jnp.min(loff[pl.ds(BPW, 16)])
        k0 = t_start // C
        k1 = (t_end + C - 1) // C

        @pl.when(k0 < k1)
        def _prime():
            pltpu.async_copy(sf_hbm.at[pl.ds(k0 * C, C)], idx.at[k0 & 1],
                             sem_idx)

        def chunk(g, _):
            p = g & 1
            s = g * C

            pltpu.make_async_copy(sf_hbm.at[pl.ds(0, C)], idx.at[p],
                                  sem_idx).wait()

            def remap(m, _):
                v = idx[p, pl.ds(m * 16, 16)]
                q = ((v >> 15) << 14) + (v & 16383)
                idx[p, pl.ds(m * 16, 16)] = 2 * q + ((v >> 14) & 1)
                return 0
            lax.fori_loop(0, C // 16, remap, 0)

            @pl.when(g > k0 + 1)
            def _drain_prev():
                pltpu.make_async_copy(rows.at[p], acc.at[seg.at[p]],
                                      sem_sca).wait()

            get = pltpu.async_copy(table_hbm.at[idx.at[p]], rows.at[p],
                                   sem_gat)

            @pl.when(g + 1 < k1)
            def _next_idx():
                pltpu.async_copy(sf_hbm.at[pl.ds((g + 1) * C, C)],
                                 idx.at[1 - p], sem_idx)

            def bsearch(m, _):
                pos = iota16 + (s + m * 16)
                res = rtzero
                step = BPW
                while step >= 1:
                    probe = res + step
                    pidx = jnp.minimum(probe, BPW + 1) - 1
                    v = plsc.load_gather(loff, [pidx])
                    take = (probe <= BPW + 1) & (v <= pos)
                    res = jnp.where(take, probe, res)
                    step //= 2
                lrow = res - 1
                out_of_range = (lrow < 0) | (lrow >= BPW)
                row = jnp.where(out_of_range, trash, lbase + lrow)
                seg[p, pl.ds(m * 16, 16)] = row
                return 0
            lax.fori_loop(0, C // 16, bsearch, 0)

            get.wait()
            pltpu.async_copy(rows.at[p], acc.at[seg.at[p]], sem_sca, add=True)
            return 0

        lax.fori_loop(k0, k1, chunk, 0)

        @pl.when(k1 > k0)
        def _drain_last():
            pltpu.make_async_copy(rows.at[(k1 - 1) & 1],
                                  acc.at[seg.at[(k1 - 1) & 1]],
                                  sem_sca).wait()

        @pl.when(k1 > k0 + 1)
        def _drain_last2():
            pltpu.make_async_copy(rows.at[k1 & 1], acc.at[seg.at[k1 & 1]],
                                  sem_sca).wait()

        for blk in range(BPW // IB):
            pltpu.sync_copy(acc.at[pl.ds(lbase + blk * IB, IB)],
                            out_hbm.at[pl.ds(gbase + blk * IB, IB)])

    return k(sparse_features, offsets_pad, table)


def _tc_pack(table_t):
    def body(lo_ref, hi_ref, o_ref):
        o_ref[:, 0:DIM] = lo_ref[...].T
        o_ref[:, DIM:2 * DIM] = hi_ref[...].T

    return pl.pallas_call(
        body,
        grid=(NPB,),
        in_specs=[
            pl.BlockSpec((DIM, PBLK), lambda i: (0, 2 * i)),
            pl.BlockSpec((DIM, PBLK),
                         lambda i: (0, jnp.minimum(2 * i + 1,
                                                   (NUM_EMB - 1) // PBLK))),
        ],
        out_specs=pl.BlockSpec((PBLK, 2 * DIM), lambda i: (i, 0)),
        out_shape=jax.ShapeDtypeStruct((NSTAGE, 2 * DIM), jnp.float32),
    )(table_t, table_t)


def _tc_transpose(table_t):
    blk = 2048

    def body(t_ref, o_ref):
        o_ref[...] = t_ref[...].T.reshape(blk // 2, 128)

    return pl.pallas_call(
        body,
        grid=((NUM_EMB + blk - 1) // blk,),
        in_specs=[pl.BlockSpec((DIM, blk), lambda i: (0, i))],
        out_specs=pl.BlockSpec((blk // 2, 128), lambda i: (i, 0)),
        out_shape=jax.ShapeDtypeStruct((NUM_EMB // 2, 2 * DIM), jnp.float32),
    )(table_t)


def _tc_finish(sums, offs0, offs1, wt_pad, b_pad):
    blk = 2048

    def body(s_ref, o0_ref, o1_ref, w_ref, b_ref, o_ref):
        cnt = jnp.maximum((o1_ref[...] - o0_ref[...]).astype(jnp.float32), 1.0)
        pooled = s_ref[...] * (1.0 / cnt)[:, None]
        o_ref[...] = (
            jnp.dot(pooled, w_ref[...], preferred_element_type=jnp.float32)
            + b_ref[...]
        )

    return pl.pallas_call(
        body,
        grid=(B // blk,),
        in_specs=[
            pl.BlockSpec((blk, DIM), lambda i: (i, 0)),
            pl.BlockSpec((blk,), lambda i: (i,)),
            pl.BlockSpec((blk,), lambda i: (i,)),
            pl.BlockSpec((DIM, 8), lambda i: (0, 0)),
            pl.BlockSpec((1, 8), lambda i: (0, 0)),
        ],
        out_specs=pl.BlockSpec((blk, 8), lambda i: (i, 0)),
        out_shape=jax.ShapeDtypeStruct((B, 8), jnp.float32),
    )(sums, offs0, offs1, wt_pad, b_pad)


def kernel(sparse_features, offsets, send_shape, table, W, b):
    offsets_pad = jnp.pad(offsets, (0, OPAD - 1), mode="edge")
    staged = _tc_pack(table.T).reshape(2 * NSTAGE, DIM)
    sums = _sc_sums(sparse_features, offsets_pad, staged)
    wt_pad = jnp.zeros((DIM, 8), jnp.float32).at[:, :5].set(W.T)
    b_pad = jnp.zeros((1, 8), jnp.float32).at[0, :5].set(b)
    out8 = _tc_finish(sums, offsets[:B], offsets[1:B + 1], wt_pad, b_pad)
    return out8[:, :5]
